# baseline (speedup 1.0000x reference)
import jax
import jax.numpy as jnp
from jax.experimental import pallas as pl
from jax.experimental.pallas import tpu as pltpu

HIDDEN = 2048
N_EXPERTS = 64
BM = 1024
NBUF = 4
NOBUF = 2


def _router_kernel(hs_hbm, w_ref, out_hbm, buf, obuf, in_sem, out_sem):
    nsteps = hs_hbm.shape[0] // BM
    w = w_ref[...]

    def in_copy(step, slot):
        return pltpu.make_async_copy(
            hs_hbm.at[pl.ds(step * BM, BM)], buf.at[slot], in_sem.at[slot]
        )

    def out_copy(step, slot):
        return pltpu.make_async_copy(
            obuf.at[slot], out_hbm.at[pl.ds(step * BM, BM)], out_sem.at[slot]
        )

    for s in range(min(NBUF, nsteps)):
        in_copy(s, s).start()

    for i in range(nsteps):
        slot = i % NBUF
        oslot = i % NOBUF
        in_copy(i, slot).wait()
        if i >= NOBUF:
            out_copy(i - NOBUF, oslot).wait()
        obuf[oslot] = jax.lax.dot_general(
            buf[slot],
            w,
            dimension_numbers=(((1,), (1,)), ((), ())),
            preferred_element_type=jnp.float32,
        )
        out_copy(i, oslot).start()
        nxt = i + NBUF
        if nxt < nsteps:
            in_copy(nxt, slot).start()

    for i in range(max(0, nsteps - NOBUF), nsteps):
        out_copy(i, i % NOBUF).wait()


def kernel(hidden_states, weight):
    hs = hidden_states.reshape(-1, HIDDEN).astype(jnp.float32)
    w = weight.astype(jnp.float32)
    m = hs.shape[0]
    return pl.pallas_call(
        _router_kernel,
        in_specs=[
            pl.BlockSpec(memory_space=pltpu.HBM),
            pl.BlockSpec(memory_space=pltpu.VMEM),
        ],
        out_specs=pl.BlockSpec(memory_space=pltpu.HBM),
        out_shape=jax.ShapeDtypeStruct((m, N_EXPERTS), jnp.float32),
        scratch_shapes=[
            pltpu.VMEM((NBUF, BM, HIDDEN), jnp.float32),
            pltpu.VMEM((NOBUF, BM, N_EXPERTS), jnp.float32),
            pltpu.SemaphoreType.DMA((NBUF,)),
            pltpu.SemaphoreType.DMA((NOBUF,)),
        ],
    )(hs, w)


# speedup vs baseline: 1.0826x; 1.0826x over previous
import jax
import jax.numpy as jnp
from jax.experimental import pallas as pl
from jax.experimental.pallas import tpu as pltpu

HIDDEN = 2048
N_EXPERTS = 64
BLOCK_M = 1024


def _router_kernel(hs_ref, w_ref, out_ref):
    out_ref[...] = jax.lax.dot_general(
        hs_ref[...],
        w_ref[...],
        dimension_numbers=(((1,), (1,)), ((), ())),
        preferred_element_type=jnp.float32,
    )


def kernel(hidden_states, weight):
    hs = hidden_states.reshape(-1, HIDDEN).astype(jnp.float32)
    w = weight.astype(jnp.float32)
    m = hs.shape[0]
    return pl.pallas_call(
        _router_kernel,
        grid=(m // BLOCK_M,),
        in_specs=[
            pl.BlockSpec((BLOCK_M, HIDDEN), lambda i: (i, 0)),
            pl.BlockSpec(memory_space=pltpu.VMEM),
        ],
        out_specs=pl.BlockSpec((BLOCK_M, N_EXPERTS), lambda i: (i, 0)),
        out_shape=jax.ShapeDtypeStruct((m, N_EXPERTS), jnp.float32),
    )(hs, w)
